# baseline (speedup 1.0000x reference)
import functools

import jax
import jax.numpy as jnp
from jax import lax
from jax.experimental import pallas as pl
from jax.experimental.pallas import tpu as pltpu
from jax.experimental.pallas import tpu_sc as plsc

_B = 16
_MAX_NODES = 4096
_N = 32768
_D = 128

_NC = 2
_NS = 16
_QROWS = _MAX_NODES // 4
_CHUNK = 256
_CHUNKZ = 192
_ZROWS_PER_TILE = _CHUNKZ // _NS
_BITS = (128, 64, 32, 16, 8, 4, 2, 1)


def _to_dense_batch_sc(x, batch):
    mesh = plsc.VectorSubcoreMesh(core_axis_name="c", subcore_axis_name="s")

    @functools.partial(
        pl.kernel,
        mesh=mesh,
        out_type=jax.ShapeDtypeStruct((_B * _MAX_NODES, _D), jnp.float32),
        scratch_types=[
            pltpu.VMEM((_N,), jnp.int32),
            pltpu.VMEM((_CHUNK, _D), jnp.float32),
            pltpu.VMEM((_CHUNK, _D), jnp.float32),
            pltpu.VMEM((_ZROWS_PER_TILE, _D), jnp.float32),
            pltpu.VMEM_SHARED((_CHUNKZ, _D), jnp.float32),
            pltpu.SemaphoreType.DMA,
            pltpu.SemaphoreType.DMA,
            pltpu.SemaphoreType.DMA,
            pltpu.SemaphoreType.DMA,
            pltpu.SemaphoreType.DMA,
            pltpu.SemaphoreType.DMA,
        ],
        compiler_params=pltpu.CompilerParams(use_tc_tiling_on_sc=False,
                                             needs_layout_passes=False),
    )
    def run(x_hbm, batch_hbm, out_hbm, batch_v, stage0, stage1, zseed_v,
            zshared, bsem, rd0, rd1, wr0, wr1, zsem):
        sid = lax.axis_index("s")
        wid = sid * _NC + lax.axis_index("c")
        b = wid // 2
        qa = (b + wid) % 2

        batch_cp = pltpu.async_copy(batch_hbm, batch_v, bsem)

        def zinit(i, carry):
            for j in range(_D // 16):
                zseed_v[i, pl.ds(j * 16, 16)] = jnp.zeros((16,), jnp.float32)
            return carry

        lax.fori_loop(0, _ZROWS_PER_TILE, zinit, 0)
        pltpu.sync_copy(
            zseed_v,
            zshared.at[pl.ds(sid * _ZROWS_PER_TILE, _ZROWS_PER_TILE)])
        plsc.subcore_barrier()
        batch_cp.wait()

        def lower_bound(v):
            def step(_, lohi):
                lo, hi = lohi
                mid = (lo + hi) // 2
                row = batch_v[pl.ds(mid * 16, 16)]
                pred = row[0] < v
                return (jnp.where(pred, mid + 1, lo),
                        jnp.where(pred, hi, mid))

            nrows = _N // 16
            r, _ = lax.fori_loop(0, 11, step,
                                 (jnp.int32(0), jnp.int32(nrows)))
            rprev = jnp.maximum(r - 1, 0)
            row = batch_v[pl.ds(rprev * 16, 16)]
            cnt_lt = jnp.sum(jnp.where(row < v, 1, 0).astype(jnp.int32))
            return jnp.where(r == 0, 0, rprev * 16 + cnt_lt)

        seg_lo = lower_bound(b)
        seg_hi = lower_bound(b + 1)
        cnt = jnp.minimum(seg_hi - seg_lo, _MAX_NODES)

        def region_params(j):
            q = qa + j * (3 - 2 * qa)
            node_lo = q * _QROWS
            valid = jnp.clip(cnt - node_lo, 0, _QROWS)
            return valid, seg_lo + node_lo, b * _MAX_NODES + node_lo

        def zfire(j, nz):
            valid, _src0, dst0 = region_params(j)
            zcnt = _QROWS - valid
            zdst0 = dst0 + valid
            nzf = zcnt // _CHUNKZ

            def zbody(i, carry):
                pltpu.async_copy(
                    zshared,
                    out_hbm.at[pl.ds(zdst0 + i * _CHUNKZ, _CHUNKZ)], zsem)
                return carry

            lax.fori_loop(0, nzf, zbody, 0)
            return nz + nzf

        nz_total = lax.fori_loop(0, 2, zfire, jnp.int32(0))

        stages = (stage0, stage1)
        rds = (rd0, rd1)
        wrs = (wr0, wr1)

        def region_body(j, carry):
            valid, src0, dst0 = region_params(j)
            nf = valid // _CHUNK
            npair = (nf + 2) // 2

            def pbody(g, carry2):
                pltpu.async_copy(
                    x_hbm.at[pl.ds(src0 + g * _CHUNK, _CHUNK)],
                    out_hbm.at[pl.ds(dst0 + g * _CHUNK, _CHUNK)], rd0)
                return carry2

            lax.fori_loop(0, nf, pbody, 0)

            def pdrain(g, carry2):
                pltpu.make_async_copy(
                    x_hbm.at[pl.ds(src0, _CHUNK)],
                    out_hbm.at[pl.ds(dst0, _CHUNK)], rd0).wait()
                return carry2

            lax.fori_loop(0, nf, pdrain, 0)

            rem = valid - nf * _CHUNK
            roff_src = src0 + nf * _CHUNK
            roff_dst = dst0 + nf * _CHUNK

            @pl.when(rem > 0)
            def _():
                src_c = jnp.minimum(roff_src, _N - _CHUNK)
                delta = roff_src - src_c
                pltpu.sync_copy(x_hbm.at[pl.ds(src_c, _CHUNK)], stage0)
                off = jnp.int32(0)
                for s in _BITS:
                    @pl.when((rem & s) != 0)
                    def _(s=s, off=off):
                        pltpu.sync_copy(
                            stage0.at[pl.ds(delta + off, s)],
                            out_hbm.at[pl.ds(roff_dst + off, s)])
                    off = off + (rem & s)

            zcnt = _QROWS - valid
            nzf = zcnt // _CHUNKZ
            zrem = zcnt - nzf * _CHUNKZ
            zoff = dst0 + valid + nzf * _CHUNKZ
            for s in _BITS:
                @pl.when((zrem & s) != 0)
                def _(s=s, zoff=zoff):
                    pltpu.sync_copy(zshared.at[pl.ds(0, s)],
                                    out_hbm.at[pl.ds(zoff, s)])
                zoff = zoff + (zrem & s)
            return carry

        lax.fori_loop(0, 2, region_body, 0)

        def zdrain(i, carry):
            pltpu.make_async_copy(
                zshared, out_hbm.at[pl.ds(0, _CHUNKZ)], zsem).wait()
            return carry

        lax.fori_loop(0, nz_total, zdrain, 0)

    return run(x, batch)


def kernel(x, batch):
    out = _to_dense_batch_sc(x, batch.astype(jnp.int32))
    return out.reshape(_B, _MAX_NODES, _D)


# speedup vs baseline: 10.5224x; 10.5224x over previous
import functools

import jax
import jax.numpy as jnp
from jax import lax
from jax.experimental import pallas as pl
from jax.experimental.pallas import tpu as pltpu
from jax.experimental.pallas import tpu_sc as plsc

_B = 16
_MAX_NODES = 4096
_N = 32768
_D = 128

_NC = 2
_NS = 16
_QROWS = _MAX_NODES // 4
_CHUNK = 256
_CHUNKZ = 512
_ZROWS_PER_TILE = _CHUNKZ // _NS
_BITS = (128, 64, 32, 16, 8, 4, 2, 1)
_ZBITS = (256,) + _BITS


def _to_dense_batch_sc(x, batch):
    mesh = plsc.VectorSubcoreMesh(core_axis_name="c", subcore_axis_name="s")

    @functools.partial(
        pl.kernel,
        mesh=mesh,
        out_type=jax.ShapeDtypeStruct((_B * _MAX_NODES, _D), jnp.float32),
        scratch_types=[
            pltpu.VMEM((_N,), jnp.int32),
            pltpu.VMEM((_CHUNK, _D), jnp.float32),
            pltpu.VMEM((_CHUNK, _D), jnp.float32),
            pltpu.VMEM((_ZROWS_PER_TILE, _D), jnp.float32),
            pltpu.VMEM_SHARED((_CHUNKZ, _D), jnp.float32),
            pltpu.SemaphoreType.DMA,
            pltpu.SemaphoreType.DMA,
            pltpu.SemaphoreType.DMA,
            pltpu.SemaphoreType.DMA,
            pltpu.SemaphoreType.DMA,
            pltpu.SemaphoreType.DMA,
        ],
        compiler_params=pltpu.CompilerParams(use_tc_tiling_on_sc=False,
                                             needs_layout_passes=False),
    )
    def run(x_hbm, batch_hbm, out_hbm, batch_v, stage0, stage1, zseed_v,
            zshared, bsem, rd0, rd1, wr0, wr1, zsem):
        sid = lax.axis_index("s")
        wid = sid * _NC + lax.axis_index("c")
        b = wid // 2
        qa = (b + wid) % 2

        batch_cp = pltpu.async_copy(batch_hbm, batch_v, bsem)

        def zinit(i, carry):
            for j in range(_D // 16):
                zseed_v[i, pl.ds(j * 16, 16)] = jnp.zeros((16,), jnp.float32)
            return carry

        lax.fori_loop(0, _ZROWS_PER_TILE, zinit, 0)
        pltpu.sync_copy(
            zseed_v,
            zshared.at[pl.ds(sid * _ZROWS_PER_TILE, _ZROWS_PER_TILE)])
        plsc.subcore_barrier()
        batch_cp.wait()

        def lower_bound(v):
            def step(_, lohi):
                lo, hi = lohi
                mid = (lo + hi) // 2
                row = batch_v[pl.ds(mid * 16, 16)]
                pred = row[0] < v
                return (jnp.where(pred, mid + 1, lo),
                        jnp.where(pred, hi, mid))

            nrows = _N // 16
            r, _ = lax.fori_loop(0, 11, step,
                                 (jnp.int32(0), jnp.int32(nrows)))
            rprev = jnp.maximum(r - 1, 0)
            row = batch_v[pl.ds(rprev * 16, 16)]
            cnt_lt = jnp.sum(jnp.where(row < v, 1, 0).astype(jnp.int32))
            return jnp.where(r == 0, 0, rprev * 16 + cnt_lt)

        seg_lo = lower_bound(b)
        seg_hi = lower_bound(b + 1)
        cnt = jnp.minimum(seg_hi - seg_lo, _MAX_NODES)

        def region_params(j):
            q = qa + j * (3 - 2 * qa)
            node_lo = q * _QROWS
            valid = jnp.clip(cnt - node_lo, 0, _QROWS)
            return valid, seg_lo + node_lo, b * _MAX_NODES + node_lo

        def zfire(j, nz):
            valid, _src0, dst0 = region_params(j)
            zcnt = _QROWS - valid
            zdst0 = dst0 + valid
            nzf = zcnt // _CHUNKZ

            def zbody(i, carry):
                pltpu.async_copy(
                    zshared,
                    out_hbm.at[pl.ds(zdst0 + i * _CHUNKZ, _CHUNKZ)], zsem)
                return carry

            lax.fori_loop(0, nzf, zbody, 0)
            return nz + nzf

        nz_total = lax.fori_loop(0, 2, zfire, jnp.int32(0))

        stages = (stage0, stage1)
        rds = (rd0, rd1)
        wrs = (wr0, wr1)

        def region_body(j, carry):
            valid, src0, dst0 = region_params(j)
            nf = valid // _CHUNK
            npair = (nf + 2) // 2

            def pbody(g, carry2):
                for hb in range(2):
                    i = 2 * g + hb
                    st, rs, ws = stages[hb], rds[hb], wrs[hb]

                    @pl.when(jnp.logical_and(i >= 2, i < nf))
                    def _():
                        pltpu.make_async_copy(
                            st, out_hbm.at[pl.ds(dst0, _CHUNK)], ws).wait()

                    @pl.when(i < nf)
                    def _():
                        pltpu.async_copy(
                            x_hbm.at[pl.ds(src0 + i * _CHUNK, _CHUNK)],
                            st, rs)

                    po = 1 - hb
                    pst, prs, pws = stages[po], rds[po], wrs[po]
                    im1 = i - 1

                    @pl.when(jnp.logical_and(im1 >= 0, im1 < nf))
                    def _():
                        pltpu.make_async_copy(
                            x_hbm.at[pl.ds(src0, _CHUNK)], pst, prs).wait()
                        pltpu.async_copy(
                            pst,
                            out_hbm.at[pl.ds(dst0 + im1 * _CHUNK, _CHUNK)],
                            pws)
                return carry2

            lax.fori_loop(0, npair, pbody, 0)

            @pl.when(nf >= 1)
            def _():
                pltpu.make_async_copy(
                    stage0, out_hbm.at[pl.ds(dst0, _CHUNK)], wr0).wait()

            @pl.when(nf >= 2)
            def _():
                pltpu.make_async_copy(
                    stage1, out_hbm.at[pl.ds(dst0, _CHUNK)], wr1).wait()

            rem = valid - nf * _CHUNK
            roff_src = src0 + nf * _CHUNK
            roff_dst = dst0 + nf * _CHUNK

            @pl.when(rem > 0)
            def _():
                src_c = jnp.minimum(roff_src, _N - _CHUNK)
                delta = roff_src - src_c
                pltpu.sync_copy(x_hbm.at[pl.ds(src_c, _CHUNK)], stage0)
                off = jnp.int32(0)
                for s in _BITS:
                    @pl.when((rem & s) != 0)
                    def _(s=s, off=off):
                        pltpu.sync_copy(
                            stage0.at[pl.ds(delta + off, s)],
                            out_hbm.at[pl.ds(roff_dst + off, s)])
                    off = off + (rem & s)

            zcnt = _QROWS - valid
            nzf = zcnt // _CHUNKZ
            zrem = zcnt - nzf * _CHUNKZ
            zoff = dst0 + valid + nzf * _CHUNKZ
            for s in _ZBITS:
                @pl.when((zrem & s) != 0)
                def _(s=s, zoff=zoff):
                    pltpu.sync_copy(zshared.at[pl.ds(0, s)],
                                    out_hbm.at[pl.ds(zoff, s)])
                zoff = zoff + (zrem & s)
            return carry

        lax.fori_loop(0, 2, region_body, 0)

        def zdrain(i, carry):
            pltpu.make_async_copy(
                zshared, out_hbm.at[pl.ds(0, _CHUNKZ)], zsem).wait()
            return carry

        lax.fori_loop(0, nz_total, zdrain, 0)

    return run(x, batch)


def kernel(x, batch):
    out = _to_dense_batch_sc(x, batch.astype(jnp.int32))
    return out.reshape(_B, _MAX_NODES, _D)


# speedup vs baseline: 10.7985x; 1.0262x over previous
import functools

import jax
import jax.numpy as jnp
from jax import lax
from jax.experimental import pallas as pl
from jax.experimental.pallas import tpu as pltpu
from jax.experimental.pallas import tpu_sc as plsc

_B = 16
_MAX_NODES = 4096
_N = 32768
_D = 128

_NC = 2
_NS = 16
_QROWS = _MAX_NODES // 4
_CHUNK = 256
_CHUNKZ = 192
_ZROWS_PER_TILE = _CHUNKZ // _NS
_BITS = (128, 64, 32, 16, 8, 4, 2, 1)


def _to_dense_batch_sc(x, batch):
    mesh = plsc.VectorSubcoreMesh(core_axis_name="c", subcore_axis_name="s")

    @functools.partial(
        pl.kernel,
        mesh=mesh,
        out_type=jax.ShapeDtypeStruct((_B * _MAX_NODES, _D), jnp.float32),
        scratch_types=[
            pltpu.VMEM((_N,), jnp.int32),
            pltpu.VMEM((_CHUNK, _D), jnp.float32),
            pltpu.VMEM((_CHUNK, _D), jnp.float32),
            pltpu.VMEM((_ZROWS_PER_TILE, _D), jnp.float32),
            pltpu.VMEM_SHARED((_CHUNKZ, _D), jnp.float32),
            pltpu.SemaphoreType.DMA,
            pltpu.SemaphoreType.DMA,
            pltpu.SemaphoreType.DMA,
            pltpu.SemaphoreType.DMA,
            pltpu.SemaphoreType.DMA,
            pltpu.SemaphoreType.DMA,
        ],
        compiler_params=pltpu.CompilerParams(use_tc_tiling_on_sc=False,
                                             needs_layout_passes=False),
    )
    def run(x_hbm, batch_hbm, out_hbm, batch_v, stage0, stage1, zseed_v,
            zshared, bsem, rd0, rd1, wr0, wr1, zsem):
        sid = lax.axis_index("s")
        wid = sid * _NC + lax.axis_index("c")
        b = wid // 2
        qa = (b + wid) % 2

        batch_cp = pltpu.async_copy(batch_hbm, batch_v, bsem)

        def zinit(i, carry):
            for j in range(_D // 16):
                zseed_v[i, pl.ds(j * 16, 16)] = jnp.zeros((16,), jnp.float32)
            return carry

        lax.fori_loop(0, _ZROWS_PER_TILE, zinit, 0)
        pltpu.sync_copy(
            zseed_v,
            zshared.at[pl.ds(sid * _ZROWS_PER_TILE, _ZROWS_PER_TILE)])
        plsc.subcore_barrier()
        batch_cp.wait()

        def lower_bound(v):
            def step(_, lohi):
                lo, hi = lohi
                mid = (lo + hi) // 2
                row = batch_v[pl.ds(mid * 16, 16)]
                pred = row[0] < v
                return (jnp.where(pred, mid + 1, lo),
                        jnp.where(pred, hi, mid))

            nrows = _N // 16
            r, _ = lax.fori_loop(0, 11, step,
                                 (jnp.int32(0), jnp.int32(nrows)))
            rprev = jnp.maximum(r - 1, 0)
            row = batch_v[pl.ds(rprev * 16, 16)]
            cnt_lt = jnp.sum(jnp.where(row < v, 1, 0).astype(jnp.int32))
            return jnp.where(r == 0, 0, rprev * 16 + cnt_lt)

        seg_lo = lower_bound(b)
        seg_hi = lower_bound(b + 1)
        cnt = jnp.minimum(seg_hi - seg_lo, _MAX_NODES)

        def region_params(j):
            q = qa + j * (3 - 2 * qa)
            node_lo = q * _QROWS
            valid = jnp.clip(cnt - node_lo, 0, _QROWS)
            return valid, seg_lo + node_lo, b * _MAX_NODES + node_lo

        def zfire(j, nz):
            valid, _src0, dst0 = region_params(j)
            zcnt = _QROWS - valid
            zdst0 = dst0 + valid
            nzf = zcnt // _CHUNKZ

            def zbody(i, carry):
                pltpu.async_copy(
                    zshared,
                    out_hbm.at[pl.ds(zdst0 + i * _CHUNKZ, _CHUNKZ)], zsem)
                return carry

            lax.fori_loop(0, nzf, zbody, 0)
            return nz + nzf

        nz_total = lax.fori_loop(0, 2, zfire, jnp.int32(0))

        stages = (stage0, stage1)
        rds = (rd0, rd1)
        wrs = (wr0, wr1)

        def region_body(j, carry):
            valid, src0, dst0 = region_params(j)
            nf = valid // _CHUNK
            npair = (nf + 2) // 2

            def pbody(g, carry2):
                for hb in range(2):
                    i = 2 * g + hb
                    st, rs, ws = stages[hb], rds[hb], wrs[hb]

                    @pl.when(jnp.logical_and(i >= 2, i < nf))
                    def _():
                        pltpu.make_async_copy(
                            st, out_hbm.at[pl.ds(dst0, _CHUNK)], ws).wait()

                    @pl.when(i < nf)
                    def _():
                        pltpu.async_copy(
                            x_hbm.at[pl.ds(src0 + i * _CHUNK, _CHUNK)],
                            st, rs)

                    po = 1 - hb
                    pst, prs, pws = stages[po], rds[po], wrs[po]
                    im1 = i - 1

                    @pl.when(jnp.logical_and(im1 >= 0, im1 < nf))
                    def _():
                        pltpu.make_async_copy(
                            x_hbm.at[pl.ds(src0, _CHUNK)], pst, prs).wait()
                        pltpu.async_copy(
                            pst,
                            out_hbm.at[pl.ds(dst0 + im1 * _CHUNK, _CHUNK)],
                            pws)
                return carry2

            lax.fori_loop(0, npair, pbody, 0)

            @pl.when(nf >= 1)
            def _():
                pltpu.make_async_copy(
                    stage0, out_hbm.at[pl.ds(dst0, _CHUNK)], wr0).wait()

            @pl.when(nf >= 2)
            def _():
                pltpu.make_async_copy(
                    stage1, out_hbm.at[pl.ds(dst0, _CHUNK)], wr1).wait()

            rem = valid - nf * _CHUNK
            roff_src = src0 + nf * _CHUNK
            roff_dst = dst0 + nf * _CHUNK

            @pl.when(rem > 0)
            def _():
                src_c = jnp.minimum(roff_src, _N - _CHUNK)
                delta = roff_src - src_c
                pltpu.sync_copy(x_hbm.at[pl.ds(src_c, _CHUNK)], stage0)
                off = jnp.int32(0)
                for s in _BITS:
                    @pl.when((rem & s) != 0)
                    def _(s=s, off=off):
                        pltpu.sync_copy(
                            stage0.at[pl.ds(delta + off, s)],
                            out_hbm.at[pl.ds(roff_dst + off, s)])
                    off = off + (rem & s)

            zcnt = _QROWS - valid
            nzf = zcnt // _CHUNKZ
            zrem = zcnt - nzf * _CHUNKZ
            zoff = dst0 + valid + nzf * _CHUNKZ
            for s in _BITS:
                @pl.when((zrem & s) != 0)
                def _(s=s, zoff=zoff):
                    pltpu.sync_copy(zshared.at[pl.ds(0, s)],
                                    out_hbm.at[pl.ds(zoff, s)])
                zoff = zoff + (zrem & s)
            return carry

        lax.fori_loop(0, 2, region_body, 0)

        def zdrain(i, carry):
            pltpu.make_async_copy(
                zshared, out_hbm.at[pl.ds(0, _CHUNKZ)], zsem).wait()
            return carry

        lax.fori_loop(0, nz_total, zdrain, 0)

    return run(x, batch)


def kernel(x, batch):
    out = _to_dense_batch_sc(x, batch.astype(jnp.int32))
    return out.reshape(_B, _MAX_NODES, _D)
